# baseline (speedup 1.0000x reference)
import functools

import jax
import jax.numpy as jnp
from jax import lax
from jax.experimental import pallas as pl
from jax.experimental.pallas import tpu as pltpu
from jax.experimental.pallas import tpu_sc as plsc

NCLS = 15
CLS_W = 2.0
BBOX_W = 5.0
ANG_W = 2.0
NOOBJ_W = 0.1
B, Q, M = 4, 1000, 200
QP, MP = 1024, 256
NPP = NCLS + 1 + 5
LANES = 16
BIGI = 2 ** 30

_COS_C = [
    1.0, -0.5, 1.0 / 24, -1.0 / 720, 1.0 / 40320, -1.0 / 3628800,
    1.0 / 479001600, -1.0 / 87178291200,
]


def _cos_scalar(x):
    t = x * x
    r = jnp.float32(_COS_C[7])
    for k in range(6, -1, -1):
        r = r * t + jnp.float32(_COS_C[k])
    return r


def _tc_body(size_ref, lg_ref, pb_ref, tb_ref, lab_ref,
             cost_ref, colmin_ref, colargq_ref, pp_ref, tg_ref, base_ref):
    zq = jnp.zeros((16, QP - Q), jnp.float32)
    lt = jnp.concatenate([lg_ref[0], zq], axis=1)
    pbt = jnp.concatenate([pb_ref[0], zq[:5]], axis=1)
    mx = jnp.max(lt, axis=0, keepdims=True)
    ex = jnp.exp(lt - mx)
    s = jnp.sum(ex, axis=0, keepdims=True)
    logp = lt - mx - jnp.log(s)
    pp_ref[0, :NCLS + 1, :] = logp
    pp_ref[0, NCLS + 1:, :] = pbt
    prob = ex / s

    hh = size_ref[0, 0, 0].astype(jnp.float32)
    ww = size_ref[0, 0, 1].astype(jnp.float32)
    bidx = pl.program_id(0)
    tbr = tb_ref[bidx]
    labr = jnp.swapaxes(lab_ref[pl.ds(bidx, 1), :], 0, 1)
    tg_ref[0, :, :5] = tbr
    tg_ref[0, :, 5:6] = labr.astype(jnp.float32)
    tg_ref[0, :, 6:] = jnp.zeros((M, 2), jnp.float32)
    tb = jnp.concatenate([tbr, jnp.zeros((MP - M, 5), jnp.float32)], axis=0)
    lab0 = jnp.concatenate(
        [labr - 1, jnp.full((MP - M, 1), NCLS, jnp.int32)], axis=0)

    sel = [prob[c:c + 1, :] for c in range(16)]
    for bit in (1, 2, 4, 8):
        cond = (lab0 & bit) != 0
        sel = [jnp.where(cond, sel[i + 1], sel[i])
               for i in range(0, len(sel), 2)]
    cls_cost = sel[0] * (-CLS_W)

    di = lax.broadcasted_iota(jnp.int32, (1, 5), 1)
    scale5 = jnp.where((di == 0) | (di == 2), ww,
                       jnp.where(di == 4, 1.0, hh))
    tbn = tb / scale5
    l1 = jnp.abs(pbt[0:1, :] - tbn[:, 0:1])
    for d in range(1, 4):
        l1 = l1 + jnp.abs(pbt[d:d + 1, :] - tbn[:, d:d + 1])
    pth = pbt[4:5, :]
    tth = tb[:, 4:5]
    ang = 1.0 - (jnp.cos(pth) * jnp.cos(tth) + jnp.sin(pth) * jnp.sin(tth))
    cost = cls_cost + l1 * BBOX_W + ang * ANG_W

    qi = lax.broadcasted_iota(jnp.int32, (MP, QP), 1)
    mi = lax.broadcasted_iota(jnp.int32, (MP, QP), 0)
    cost = jnp.where((qi >= Q) | (mi >= M), jnp.inf, cost)
    cost_ref[0] = cost[:M]

    cmin = jnp.min(cost, axis=1, keepdims=True)
    colmin_ref[0] = jnp.swapaxes(cmin, 0, 1)
    ismin = cost == cmin
    argq = jnp.min(jnp.where(ismin, qi, QP), axis=1, keepdims=True)
    colargq_ref[0] = jnp.swapaxes(argq, 0, 1)

    row15 = logp[NCLS:NCLS + 1, :]
    qrow = lax.broadcasted_iota(jnp.int32, (1, QP), 1)
    base_ref[0, 0, 0] = NOOBJ_W * jnp.sum(jnp.where(qrow < Q, -row15, 0.0))


def _tc_stage(size, lg, pb, tbr, labr):
    f32 = jnp.float32
    out_shapes = (
        jax.ShapeDtypeStruct((B, M, QP), f32),
        jax.ShapeDtypeStruct((B, 1, MP), f32),
        jax.ShapeDtypeStruct((B, 1, MP), jnp.int32),
        jax.ShapeDtypeStruct((B, NPP, QP), f32),
        jax.ShapeDtypeStruct((B, M, 8), f32),
        jax.ShapeDtypeStruct((B, 1, 1), f32),
    )
    grid = (B,)
    return pl.pallas_call(
        _tc_body,
        grid=grid,
        in_specs=[
            pl.BlockSpec((1, 1, 2), lambda b: (b, 0, 0), memory_space=pltpu.SMEM),
            pl.BlockSpec((1, 16, Q), lambda b: (b, 0, 0)),
            pl.BlockSpec((1, 5, Q), lambda b: (b, 0, 0)),
            pl.BlockSpec((B, M, 5), lambda b: (0, 0, 0)),
            pl.BlockSpec((B, M), lambda b: (0, 0)),
        ],
        out_specs=[
            pl.BlockSpec((1, M, QP), lambda b: (b, 0, 0)),
            pl.BlockSpec((1, 1, MP), lambda b: (b, 0, 0)),
            pl.BlockSpec((1, 1, MP), lambda b: (b, 0, 0)),
            pl.BlockSpec((1, NPP, QP), lambda b: (b, 0, 0)),
            pl.BlockSpec((1, M, 8), lambda b: (b, 0, 0)),
            pl.BlockSpec((1, 1, 1), lambda b: (b, 0, 0), memory_space=pltpu.SMEM),
        ],
        out_shape=out_shapes,
    )(size, lg, pb, tbr, labr)


def _sc_greedy(cost_hbm, colmin_hbm, colargq_hbm, pp_hbm, tg_hbm, size_hbm,
               out_hbm,
               colmin_v, colargq_v, pp_v, tg_v, size_v,
               qmask_v, rowbuf_v, outbuf_v, summary_v, mqm_v, cost_sh, dsem,
               dsem2):
    info = plsc.get_sparse_core_info()
    ns = info.num_subcores
    wid = lax.axis_index("c") * ns + lax.axis_index("s")

    iota16 = lax.broadcasted_iota(jnp.int32, (LANES,), 0)
    lane0 = iota16 == 0

    def _gat(ref, *idx):
        idxs = [jnp.broadcast_to(i, (LANES,)).astype(jnp.int32) for i in idx]
        return plsc.load_gather(ref, idxs)[0]

    def _gatv(ref, idx16):
        return plsc.load_gather(ref, [idx16])

    def _put(ref, i, val):
        ii = jnp.broadcast_to(i, (LANES,)).astype(jnp.int32)
        plsc.store_scatter(ref, [ii], jnp.broadcast_to(val, (LANES,)),
                           mask=lane0)

    @pl.when(wid < B)
    def _work():
        b = wid
        with jax.named_scope("sc_stage_in"):
            stage = pltpu.async_copy(cost_hbm.at[b], cost_sh.at[b], dsem)
            hs = [pltpu.async_copy(colmin_hbm.at[b, 0], colmin_v, dsem2),
                  pltpu.async_copy(colargq_hbm.at[b, 0], colargq_v, dsem2),
                  pltpu.async_copy(pp_hbm.at[b], pp_v, dsem2),
                  pltpu.async_copy(tg_hbm.at[b], tg_v, dsem2),
                  pltpu.async_copy(size_hbm.at[b], size_v, dsem2)]
            for h in hs:
                h.wait()

        zeros16 = jnp.zeros((LANES,), jnp.float32)
        for k in range(QP // LANES):
            qmask_v[pl.ds(k * LANES, LANES)] = zeros16
        for k in range(MP // LANES):
            _put(summary_v, k, jnp.min(colmin_v[pl.ds(k * LANES, LANES)]))
        mqm_v[pl.ds(M - 8, LANES)] = jnp.zeros((LANES,), jnp.int32)

        sizes = size_v[pl.ds(0, LANES)]
        rcp = 1.0 / sizes.astype(jnp.float32)
        rw = rcp[1]
        rh = rcp[0]
        rs = (rw, rh, rw, rh)
        inf = jnp.float32(jnp.inf)

        def upd_summary(m):
            k = lax.shift_right_logical(m, 4)
            _put(summary_v, k, jnp.min(colmin_v[pl.ds(k * LANES, LANES)]))

        def recompute_col(m2):
            pltpu.sync_copy(cost_sh.at[b, m2], rowbuf_v)
            bv = rowbuf_v[pl.ds(0, LANES)] + qmask_v[pl.ds(0, LANES)]
            bq = iota16
            for k in range(1, QP // LANES):
                v = rowbuf_v[pl.ds(k * LANES, LANES)] + qmask_v[pl.ds(k * LANES, LANES)]
                qv = iota16 + (k * LANES)
                lt2 = (v < bv) | ((v == bv) & (qv < bq))
                bv = jnp.where(lt2, v, bv)
                bq = jnp.where(lt2, qv, bq)
            mv = jnp.min(bv)
            _put(colmin_v, m2, mv)
            _put(colargq_v, m2, jnp.min(jnp.where(bv == mv, bq, BIGI)))
            upd_summary(m2)

        def full_scan():
            bv = colmin_v[pl.ds(0, LANES)]
            bq = colargq_v[pl.ds(0, LANES)]
            bm = iota16
            for k in range(1, MP // LANES):
                v = colmin_v[pl.ds(k * LANES, LANES)]
                qv = colargq_v[pl.ds(k * LANES, LANES)]
                mv_ = iota16 + (k * LANES)
                lt2 = (v < bv) | ((v == bv) & ((qv < bq) | ((qv == bq) & (mv_ < bm))))
                bv = jnp.where(lt2, v, bv)
                bq = jnp.where(lt2, qv, bq)
                bm = jnp.where(lt2, mv_, bm)
            gv = jnp.min(bv)
            c1 = bv == gv
            gq = jnp.min(jnp.where(c1, bq, BIGI))
            gm = jnp.min(jnp.where(c1 & (bq == gq), bm, BIGI))
            return gv, gq, gm

        def full_scan_alt():
            gv, gq, gm = full_scan()
            ks = lax.shift_right_logical(gm, 4)
            cv = colmin_v[pl.ds(ks * LANES, LANES)]
            cv = jnp.where(iota16 == (gm & (LANES - 1)), jnp.inf, cv)
            return gv, gq, gm, jnp.min(cv)

        def scan_raw():
            sv = summary_v[pl.ds(0, LANES)]
            sk, skidx = plsc.sort_key_val(sv, iota16)
            k0 = skidx[0]
            off = k0 * LANES
            v = colmin_v[pl.ds(off, LANES)]
            qv = colargq_v[pl.ds(off, LANES)]
            ck, cp = plsc.sort_key_val(v, qv * LANES + iota16)
            p0 = cp[0]
            tie = (sk[1] == sk[0]) | (ck[1] == ck[0])
            return (tie, ck[0], lax.shift_right_logical(p0, 4),
                    off + (p0 & (LANES - 1)), ck[1])

        def body(it, carry):
            tie, v0, q0, m0, a0 = scan_raw()
            stale = _gat(qmask_v, q0) != 0.0

            def slowpath():
                st = lax.cond(tie, full_scan_alt,
                              lambda: (v0, q0, m0, a0))

                def wcond(s):
                    return _gat(qmask_v, s[1]) != 0.0

                def wbody(s):
                    recompute_col(s[2])
                    t2, v2, q2, m2, a2 = scan_raw()
                    return lax.cond(t2, full_scan_alt,
                                    lambda: (v2, q2, m2, a2))

                return lax.while_loop(wcond, wbody, st)

            gv, gq, gm, alt = lax.cond(tie | stale, slowpath,
                                       lambda: (v0, q0, m0, a0))

            _put(mqm_v, it, gq * MP + gm)

            _put(colmin_v, gm, inf)
            _put(qmask_v, gq, inf)
            _put(summary_v, lax.shift_right_logical(gm, 4), alt)
            return carry

        with jax.named_scope("sc_match"):
            stage.wait()
            lax.fori_loop(0, M, body, 0, unroll=4)

        z16 = jnp.zeros((LANES,), jnp.float32)
        ccorr_v = z16
        bsum_v = z16
        asum_v = z16
        for g in range(MP // LANES):
            base_i = g * LANES
            if base_i >= M:
                break
            pk = mqm_v[pl.ds(base_i, LANES)]
            q16 = lax.shift_right_logical(pk, 8)
            m16 = pk & (MP - 1)
            c5 = jnp.broadcast_to(jnp.int32(5), (LANES,))
            labm16 = plsc.load_gather(tg_v, [m16, c5]).astype(jnp.int32) - 1
            lp = plsc.load_gather(pp_v, [labm16, q16])
            lp15 = plsc.load_gather(pp_v, [c5 + (NCLS - 5), q16])
            t = [plsc.load_gather(tg_v, [m16, jnp.broadcast_to(jnp.int32(d), (LANES,))])
                 for d in range(5)]
            p = [plsc.load_gather(pp_v, [jnp.broadcast_to(jnp.int32(NCLS + 1 + d), (LANES,)), q16])
                 for d in range(5)]
            cc = -lp + NOOBJ_W * lp15
            l1 = jnp.abs(p[0] - t[0] * rs[0])
            for d in range(1, 4):
                l1 = l1 + jnp.abs(p[d] - t[d] * rs[d])
            dth = p[4] - t[4]
            av = 1.0 - _cos_scalar(dth)
            if base_i + LANES > M:
                valid = iota16 < (M - base_i)
                cc = jnp.where(valid, cc, 0.0)
                l1 = jnp.where(valid, l1, 0.0)
                av = jnp.where(valid, av, 0.0)
            ccorr_v = ccorr_v + cc
            bsum_v = bsum_v + l1
            asum_v = asum_v + av
        ccorr = jnp.sum(ccorr_v)
        bsum = jnp.sum(bsum_v)
        asum = jnp.sum(asum_v)
        out16 = jnp.where(iota16 == 0, ccorr,
                          jnp.where(iota16 == 1, bsum,
                                    jnp.where(iota16 == 2, asum, 0.0)))
        outbuf_v[pl.ds(0, LANES)] = out16
        pltpu.sync_copy(outbuf_v, out_hbm.at[b])


def _sc_stage(cost, colmin, colargq, pp, tg, size):
    mesh = plsc.VectorSubcoreMesh(core_axis_name="c", subcore_axis_name="s")
    f32 = jnp.float32
    fn = functools.partial(
        pl.kernel,
        mesh=mesh,
        compiler_params=pltpu.CompilerParams(needs_layout_passes=False),
        out_type=jax.ShapeDtypeStruct((B, 16), f32),
        scratch_types=[
            pltpu.VMEM((MP,), f32),
            pltpu.VMEM((MP,), jnp.int32),
            pltpu.VMEM((NPP, QP), f32),
            pltpu.VMEM((M, 8), f32),
            pltpu.VMEM((LANES,), jnp.int32),
            pltpu.VMEM((QP,), f32),
            pltpu.VMEM((QP,), f32),
            pltpu.VMEM((LANES,), f32),
            pltpu.VMEM((LANES,), f32),
            pltpu.VMEM((MP,), jnp.int32),
            pltpu.VMEM_SHARED((B, M, QP), f32),
            pltpu.SemaphoreType.DMA,
            pltpu.SemaphoreType.DMA,
        ],
    )(_sc_greedy)
    return fn(cost, colmin, colargq, pp, tg, size)


@jax.jit
def kernel(pred_logits, pred_boxes, tgt_boxes, tgt_labels, tgt_size):
    f32 = jnp.float32
    size32 = tgt_size.astype(jnp.int32)
    sizep = jnp.pad(size32, ((0, 0), (0, 14)))

    cost, colmin, colargq, pp, tg, base = _tc_stage(
        size32[:, None, :],
        jnp.swapaxes(pred_logits.astype(f32), 1, 2),
        jnp.swapaxes(pred_boxes.astype(f32), 1, 2),
        tgt_boxes.astype(f32), tgt_labels.astype(jnp.int32))

    res = _sc_stage(cost, colmin, colargq, pp, tg, sizep)

    denom = NOOBJ_W * (Q - M) + 1.0 * M
    loss_cls = jnp.mean((base[:, 0, 0] + res[:, 0]) / denom)
    loss_bbox = jnp.mean(res[:, 1] / (M * 4)) * BBOX_W
    loss_ang = jnp.mean(res[:, 2] / M) * ANG_W
    return (loss_cls + loss_bbox + loss_ang, loss_cls, loss_bbox, loss_ang)


# speedup vs baseline: 1.0167x; 1.0167x over previous
import functools

import jax
import jax.numpy as jnp
from jax import lax
from jax.experimental import pallas as pl
from jax.experimental.pallas import tpu as pltpu
from jax.experimental.pallas import tpu_sc as plsc

NCLS = 15
CLS_W = 2.0
BBOX_W = 5.0
ANG_W = 2.0
NOOBJ_W = 0.1
B, Q, M = 4, 1000, 200
QP, MP = 1024, 256
NPP = NCLS + 1 + 5
LANES = 16
BIGI = 2 ** 30

_COS_C = [
    1.0, -0.5, 1.0 / 24, -1.0 / 720, 1.0 / 40320, -1.0 / 3628800,
    1.0 / 479001600, -1.0 / 87178291200,
]


def _cos_scalar(x):
    t = x * x
    r = jnp.float32(_COS_C[7])
    for k in range(6, -1, -1):
        r = r * t + jnp.float32(_COS_C[k])
    return r


def _tc_body(size_ref, lg_ref, pb_ref, tb_ref, lab_ref,
             cost_ref, colmin_ref, colargq_ref, pp_ref, tg_ref, base_ref):
    zq = jnp.zeros((16, QP - Q), jnp.float32)
    lt = jnp.concatenate([lg_ref[0], zq], axis=1)
    pbt = jnp.concatenate([pb_ref[0], zq[:5]], axis=1)
    mx = jnp.max(lt, axis=0, keepdims=True)
    ex = jnp.exp(lt - mx)
    s = jnp.sum(ex, axis=0, keepdims=True)
    logp = lt - mx - jnp.log(s)
    pp_ref[0, :NCLS + 1, :] = logp
    pp_ref[0, NCLS + 1:, :] = pbt
    prob = ex / s

    hh = size_ref[0, 0, 0].astype(jnp.float32)
    ww = size_ref[0, 0, 1].astype(jnp.float32)
    bidx = pl.program_id(0)
    tbr = tb_ref[bidx]
    labr = jnp.swapaxes(lab_ref[pl.ds(bidx, 1), :], 0, 1)
    tg_ref[0, :, :5] = tbr
    tg_ref[0, :, 5:6] = labr.astype(jnp.float32)
    tg_ref[0, :, 6:] = jnp.zeros((M, 2), jnp.float32)
    tb = jnp.concatenate([tbr, jnp.zeros((MP - M, 5), jnp.float32)], axis=0)
    lab0 = jnp.concatenate(
        [labr - 1, jnp.full((MP - M, 1), NCLS, jnp.int32)], axis=0)

    sel = [prob[c:c + 1, :] for c in range(16)]
    for bit in (1, 2, 4, 8):
        cond = (lab0 & bit) != 0
        sel = [jnp.where(cond, sel[i + 1], sel[i])
               for i in range(0, len(sel), 2)]
    cls_cost = sel[0] * (-CLS_W)

    di = lax.broadcasted_iota(jnp.int32, (1, 5), 1)
    scale5 = jnp.where((di == 0) | (di == 2), ww,
                       jnp.where(di == 4, 1.0, hh))
    tbn = tb / scale5
    l1 = jnp.abs(pbt[0:1, :] - tbn[:, 0:1])
    for d in range(1, 4):
        l1 = l1 + jnp.abs(pbt[d:d + 1, :] - tbn[:, d:d + 1])
    pth = pbt[4:5, :]
    tth = tb[:, 4:5]
    ang = 1.0 - (jnp.cos(pth) * jnp.cos(tth) + jnp.sin(pth) * jnp.sin(tth))
    cost = cls_cost + l1 * BBOX_W + ang * ANG_W

    qi = lax.broadcasted_iota(jnp.int32, (MP, QP), 1)
    mi = lax.broadcasted_iota(jnp.int32, (MP, QP), 0)
    cost = jnp.where((qi >= Q) | (mi >= M), jnp.inf, cost)
    cost_ref[0] = cost[:M]

    cmin = jnp.min(cost, axis=1, keepdims=True)
    colmin_ref[0] = jnp.swapaxes(cmin, 0, 1)
    ismin = cost == cmin
    argq = jnp.min(jnp.where(ismin, qi, QP), axis=1, keepdims=True)
    colargq_ref[0] = jnp.swapaxes(argq, 0, 1)

    row15 = logp[NCLS:NCLS + 1, :]
    qrow = lax.broadcasted_iota(jnp.int32, (1, QP), 1)
    base_ref[0, 0, 0] = NOOBJ_W * jnp.sum(jnp.where(qrow < Q, -row15, 0.0))


def _tc_stage(size, lg, pb, tbr, labr):
    f32 = jnp.float32
    out_shapes = (
        jax.ShapeDtypeStruct((B, M, QP), f32),
        jax.ShapeDtypeStruct((B, 1, MP), f32),
        jax.ShapeDtypeStruct((B, 1, MP), jnp.int32),
        jax.ShapeDtypeStruct((B, NPP, QP), f32),
        jax.ShapeDtypeStruct((B, M, 8), f32),
        jax.ShapeDtypeStruct((B, 1, 1), f32),
    )
    grid = (B,)
    return pl.pallas_call(
        _tc_body,
        grid=grid,
        in_specs=[
            pl.BlockSpec((1, 1, 2), lambda b: (b, 0, 0), memory_space=pltpu.SMEM),
            pl.BlockSpec((1, 16, Q), lambda b: (b, 0, 0)),
            pl.BlockSpec((1, 5, Q), lambda b: (b, 0, 0)),
            pl.BlockSpec((B, M, 5), lambda b: (0, 0, 0)),
            pl.BlockSpec((B, M), lambda b: (0, 0)),
        ],
        out_specs=[
            pl.BlockSpec((1, M, QP), lambda b: (b, 0, 0)),
            pl.BlockSpec((1, 1, MP), lambda b: (b, 0, 0)),
            pl.BlockSpec((1, 1, MP), lambda b: (b, 0, 0)),
            pl.BlockSpec((1, NPP, QP), lambda b: (b, 0, 0)),
            pl.BlockSpec((1, M, 8), lambda b: (b, 0, 0)),
            pl.BlockSpec((1, 1, 1), lambda b: (b, 0, 0), memory_space=pltpu.SMEM),
        ],
        out_shape=out_shapes,
    )(size, lg, pb, tbr, labr)


def _sc_greedy(cost_hbm, colmin_hbm, colargq_hbm, pp_hbm, tg_hbm, size_hbm,
               out_hbm,
               colmin_v, colargq_v, pp_v, tg_v, size_v,
               qmask_v, rowbuf_v, outbuf_v, summary_v, mqm_v, cost_sh, dsem,
               dsem2):
    info = plsc.get_sparse_core_info()
    ns = info.num_subcores
    wid = lax.axis_index("c") * ns + lax.axis_index("s")

    iota16 = lax.broadcasted_iota(jnp.int32, (LANES,), 0)
    lane0 = iota16 == 0

    def _gat(ref, *idx):
        idxs = [jnp.broadcast_to(i, (LANES,)).astype(jnp.int32) for i in idx]
        return plsc.load_gather(ref, idxs)[0]

    def _gatv(ref, idx16):
        return plsc.load_gather(ref, [idx16])

    def _put(ref, i, val):
        ii = jnp.broadcast_to(i, (LANES,)).astype(jnp.int32)
        plsc.store_scatter(ref, [ii], jnp.broadcast_to(val, (LANES,)),
                           mask=lane0)

    @pl.when(wid < B)
    def _work():
        b = wid
        with jax.named_scope("sc_stage_in"):
            stage = pltpu.async_copy(cost_hbm.at[b], cost_sh.at[b], dsem)
            hs = [pltpu.async_copy(colmin_hbm.at[b, 0], colmin_v, dsem2),
                  pltpu.async_copy(colargq_hbm.at[b, 0], colargq_v, dsem2),
                  pltpu.async_copy(pp_hbm.at[b], pp_v, dsem2),
                  pltpu.async_copy(tg_hbm.at[b], tg_v, dsem2),
                  pltpu.async_copy(size_hbm.at[b], size_v, dsem2)]
            for h in hs:
                h.wait()

        zeros16 = jnp.zeros((LANES,), jnp.float32)
        for k in range(QP // LANES):
            qmask_v[pl.ds(k * LANES, LANES)] = zeros16
        for k in range(MP // LANES):
            _put(summary_v, k, jnp.min(colmin_v[pl.ds(k * LANES, LANES)]))
        mqm_v[pl.ds(M - 8, LANES)] = jnp.zeros((LANES,), jnp.int32)

        sizes = size_v[pl.ds(0, LANES)]
        rcp = 1.0 / sizes.astype(jnp.float32)
        rw = rcp[1]
        rh = rcp[0]
        rs = (rw, rh, rw, rh)
        inf = jnp.float32(jnp.inf)

        def upd_summary(m):
            k = lax.shift_right_logical(m, 4)
            _put(summary_v, k, jnp.min(colmin_v[pl.ds(k * LANES, LANES)]))

        def recompute_col(m2):
            pltpu.sync_copy(cost_sh.at[b, m2], rowbuf_v)
            bv = rowbuf_v[pl.ds(0, LANES)] + qmask_v[pl.ds(0, LANES)]
            bq = iota16
            for k in range(1, QP // LANES):
                v = rowbuf_v[pl.ds(k * LANES, LANES)] + qmask_v[pl.ds(k * LANES, LANES)]
                qv = iota16 + (k * LANES)
                lt2 = (v < bv) | ((v == bv) & (qv < bq))
                bv = jnp.where(lt2, v, bv)
                bq = jnp.where(lt2, qv, bq)
            mv = jnp.min(bv)
            _put(colmin_v, m2, mv)
            _put(colargq_v, m2, jnp.min(jnp.where(bv == mv, bq, BIGI)))
            upd_summary(m2)

        def full_scan():
            bv = colmin_v[pl.ds(0, LANES)]
            bq = colargq_v[pl.ds(0, LANES)]
            bm = iota16
            for k in range(1, MP // LANES):
                v = colmin_v[pl.ds(k * LANES, LANES)]
                qv = colargq_v[pl.ds(k * LANES, LANES)]
                mv_ = iota16 + (k * LANES)
                lt2 = (v < bv) | ((v == bv) & ((qv < bq) | ((qv == bq) & (mv_ < bm))))
                bv = jnp.where(lt2, v, bv)
                bq = jnp.where(lt2, qv, bq)
                bm = jnp.where(lt2, mv_, bm)
            gv = jnp.min(bv)
            c1 = bv == gv
            gq = jnp.min(jnp.where(c1, bq, BIGI))
            gm = jnp.min(jnp.where(c1 & (bq == gq), bm, BIGI))
            return gv, gq, gm

        def full_scan_alt():
            gv, gq, gm = full_scan()
            ks = lax.shift_right_logical(gm, 4)
            cv = colmin_v[pl.ds(ks * LANES, LANES)]
            cv = jnp.where(iota16 == (gm & (LANES - 1)), jnp.inf, cv)
            return gv, gq, gm, jnp.min(cv)

        def scan_raw():
            sv = summary_v[pl.ds(0, LANES)]
            sk, skidx = plsc.sort_key_val(sv, iota16)
            k0 = skidx[0]
            off = k0 * LANES
            v = colmin_v[pl.ds(off, LANES)]
            qv = colargq_v[pl.ds(off, LANES)]
            ck, cp = plsc.sort_key_val(v, qv * LANES + iota16)
            p0 = cp[0]
            tie = (sk[1] == sk[0]) | (ck[1] == ck[0])
            return (tie, ck[0], lax.shift_right_logical(p0, 4),
                    off + (p0 & (LANES - 1)), ck[1])

        def body(it, carry):
            tie, v0, q0, m0, a0 = scan_raw()
            stale = _gat(qmask_v, q0) != 0.0

            def slowpath():
                st = lax.cond(tie, full_scan_alt,
                              lambda: (v0, q0, m0, a0))

                def wcond(s):
                    return _gat(qmask_v, s[1]) != 0.0

                def wbody(s):
                    recompute_col(s[2])
                    t2, v2, q2, m2, a2 = scan_raw()
                    return lax.cond(t2, full_scan_alt,
                                    lambda: (v2, q2, m2, a2))

                return lax.while_loop(wcond, wbody, st)

            gv, gq, gm, alt = lax.cond(tie | stale, slowpath,
                                       lambda: (v0, q0, m0, a0))

            _put(mqm_v, it, gq * MP + gm)

            _put(colmin_v, gm, inf)
            _put(qmask_v, gq, inf)
            _put(summary_v, lax.shift_right_logical(gm, 4), alt)
            return carry

        with jax.named_scope("sc_match"):
            stage.wait()
            lax.fori_loop(0, M, body, 0, unroll=2)

        z16 = jnp.zeros((LANES,), jnp.float32)
        ccorr_v = z16
        bsum_v = z16
        asum_v = z16
        for g in range(MP // LANES):
            base_i = g * LANES
            if base_i >= M:
                break
            pk = mqm_v[pl.ds(base_i, LANES)]
            q16 = lax.shift_right_logical(pk, 8)
            m16 = pk & (MP - 1)
            c5 = jnp.broadcast_to(jnp.int32(5), (LANES,))
            labm16 = plsc.load_gather(tg_v, [m16, c5]).astype(jnp.int32) - 1
            lp = plsc.load_gather(pp_v, [labm16, q16])
            lp15 = plsc.load_gather(pp_v, [c5 + (NCLS - 5), q16])
            t = [plsc.load_gather(tg_v, [m16, jnp.broadcast_to(jnp.int32(d), (LANES,))])
                 for d in range(5)]
            p = [plsc.load_gather(pp_v, [jnp.broadcast_to(jnp.int32(NCLS + 1 + d), (LANES,)), q16])
                 for d in range(5)]
            cc = -lp + NOOBJ_W * lp15
            l1 = jnp.abs(p[0] - t[0] * rs[0])
            for d in range(1, 4):
                l1 = l1 + jnp.abs(p[d] - t[d] * rs[d])
            dth = p[4] - t[4]
            av = 1.0 - _cos_scalar(dth)
            if base_i + LANES > M:
                valid = iota16 < (M - base_i)
                cc = jnp.where(valid, cc, 0.0)
                l1 = jnp.where(valid, l1, 0.0)
                av = jnp.where(valid, av, 0.0)
            ccorr_v = ccorr_v + cc
            bsum_v = bsum_v + l1
            asum_v = asum_v + av
        ccorr = jnp.sum(ccorr_v)
        bsum = jnp.sum(bsum_v)
        asum = jnp.sum(asum_v)
        out16 = jnp.where(iota16 == 0, ccorr,
                          jnp.where(iota16 == 1, bsum,
                                    jnp.where(iota16 == 2, asum, 0.0)))
        outbuf_v[pl.ds(0, LANES)] = out16
        pltpu.sync_copy(outbuf_v, out_hbm.at[b])


def _sc_stage(cost, colmin, colargq, pp, tg, size):
    mesh = plsc.VectorSubcoreMesh(core_axis_name="c", subcore_axis_name="s")
    f32 = jnp.float32
    fn = functools.partial(
        pl.kernel,
        mesh=mesh,
        compiler_params=pltpu.CompilerParams(needs_layout_passes=False),
        out_type=jax.ShapeDtypeStruct((B, 16), f32),
        scratch_types=[
            pltpu.VMEM((MP,), f32),
            pltpu.VMEM((MP,), jnp.int32),
            pltpu.VMEM((NPP, QP), f32),
            pltpu.VMEM((M, 8), f32),
            pltpu.VMEM((LANES,), jnp.int32),
            pltpu.VMEM((QP,), f32),
            pltpu.VMEM((QP,), f32),
            pltpu.VMEM((LANES,), f32),
            pltpu.VMEM((LANES,), f32),
            pltpu.VMEM((MP,), jnp.int32),
            pltpu.VMEM_SHARED((B, M, QP), f32),
            pltpu.SemaphoreType.DMA,
            pltpu.SemaphoreType.DMA,
        ],
    )(_sc_greedy)
    return fn(cost, colmin, colargq, pp, tg, size)


@jax.jit
def kernel(pred_logits, pred_boxes, tgt_boxes, tgt_labels, tgt_size):
    f32 = jnp.float32
    size32 = tgt_size.astype(jnp.int32)
    sizep = jnp.pad(size32, ((0, 0), (0, 14)))

    cost, colmin, colargq, pp, tg, base = _tc_stage(
        size32[:, None, :],
        jnp.swapaxes(pred_logits.astype(f32), 1, 2),
        jnp.swapaxes(pred_boxes.astype(f32), 1, 2),
        tgt_boxes.astype(f32), tgt_labels.astype(jnp.int32))

    res = _sc_stage(cost, colmin, colargq, pp, tg, sizep)

    denom = NOOBJ_W * (Q - M) + 1.0 * M
    loss_cls = jnp.mean((base[:, 0, 0] + res[:, 0]) / denom)
    loss_bbox = jnp.mean(res[:, 1] / (M * 4)) * BBOX_W
    loss_ang = jnp.mean(res[:, 2] / M) * ANG_W
    return (loss_cls + loss_bbox + loss_ang, loss_cls, loss_bbox, loss_ang)
